# baseline (speedup 1.0000x reference)
import functools

import jax
import jax.numpy as jnp
from jax.experimental import pallas as pl
from jax.experimental.pallas import tpu as pltpu

_WORLD_SIZE = 8
_RANK = 0
_MARGIN = 1.0
_ROW_BLOCK = 128
_COL_BLOCK = 512
_LANES = 128


def _triplet_body(x_ref, y_ref, out_ref, pos_ref, neg_ref, *,
                  cluster_size, n_col_blocks):
    i = pl.program_id(0)
    j = pl.program_id(1)
    x = x_ref[...]
    y = y_ref[...]
    y2 = jnp.sum(y * y, axis=1)
    m = jax.lax.dot_general(
        x, y, (((1,), (1,)), ((), ())), preferred_element_type=jnp.float32
    )
    t = y2[None, :] - 2.0 * m

    inf = jnp.float32(jnp.inf)

    @pl.when(j == 0)
    def _():
        pos_ref[...] = jnp.full((_ROW_BLOCK, _LANES), -inf, jnp.float32)
        neg_ref[...] = jnp.full((_ROW_BLOCK, _LANES), inf, jnp.float32)

    r_cl = jax.lax.broadcasted_iota(jnp.int32, (_ROW_BLOCK, _LANES), 0) // cluster_size
    c_cl = jax.lax.broadcasted_iota(jnp.int32, (_ROW_BLOCK, _LANES), 1) // cluster_size
    mask16 = r_cl == c_cl

    pos_acc = pos_ref[...]
    neg_acc = neg_ref[...]
    n_sub = _COL_BLOCK // _LANES
    for k in range(n_sub):
        chunk = t[:, k * _LANES:(k + 1) * _LANES]
        is_band = (j * n_sub + k) == i
        band_mask = jnp.logical_and(is_band, mask16)
        neg_acc = jnp.minimum(neg_acc, jnp.where(band_mask, inf, chunk))
        pos_acc = jnp.maximum(pos_acc, jnp.where(band_mask, chunk, -inf))
    pos_ref[...] = pos_acc
    neg_ref[...] = neg_acc

    @pl.when(jnp.logical_and(i == 0, j == 0))
    def _():
        out_ref[...] = jnp.zeros((1, 1), jnp.float32)

    @pl.when(j == n_col_blocks - 1)
    def _():
        x2 = jnp.sum(x * x, axis=1, keepdims=True)
        pos = jnp.max(pos_acc, axis=1, keepdims=True)
        neg = jnp.min(neg_acc, axis=1, keepdims=True)
        ap = jnp.sqrt(jnp.maximum(x2 + pos, 0.0))
        an = jnp.sqrt(jnp.maximum(x2 + neg, 0.0))
        part = jnp.sum(jnp.maximum(ap - an + _MARGIN, 0.0)).reshape(1, 1)
        out_ref[...] += part


def kernel(batch):
    n_clusters, cluster_size, edim = batch.shape
    all_embeds = batch.reshape(-1, edim)
    n_total = all_embeds.shape[0]
    base, rem = divmod(n_clusters, _WORLD_SIZE)
    my_clusters = base + (1 if _RANK < rem else 0)
    my_rows = my_clusters * cluster_size
    n_row_blocks = my_rows // _ROW_BLOCK
    n_col_blocks = n_total // _COL_BLOCK

    out = pl.pallas_call(
        functools.partial(_triplet_body, cluster_size=cluster_size,
                          n_col_blocks=n_col_blocks),
        grid=(n_row_blocks, n_col_blocks),
        in_specs=[
            pl.BlockSpec((_ROW_BLOCK, edim), lambda i, j: (i, 0)),
            pl.BlockSpec((_COL_BLOCK, edim), lambda i, j: (j, 0)),
        ],
        out_specs=pl.BlockSpec((1, 1), lambda i, j: (0, 0)),
        out_shape=jax.ShapeDtypeStruct((1, 1), jnp.float32),
        scratch_shapes=[
            pltpu.VMEM((_ROW_BLOCK, _LANES), jnp.float32),
            pltpu.VMEM((_ROW_BLOCK, _LANES), jnp.float32),
        ],
        compiler_params=pltpu.CompilerParams(
            dimension_semantics=("arbitrary", "arbitrary"),
        ),
    )(all_embeds, all_embeds)
    return out[0, 0] / my_rows


# speedup vs baseline: 135.1915x; 135.1915x over previous
import functools

import jax
import jax.numpy as jnp
from jax.experimental import pallas as pl
from jax.experimental.pallas import tpu as pltpu

_WORLD_SIZE = 8
_RANK = 0
_MARGIN = 1.0
_COL_BLOCK = 512
_LANES = 128


def _triplet_body(x_ref, y_ref, out_ref, xs_ref, pos_ref, neg_ref, *,
                  rows, edim, cluster_size, n_col_blocks, n_band_blocks):
    j = pl.program_id(0)
    inf = jnp.float32(jnp.inf)

    @pl.when(j == 0)
    def _():
        xs_ref[...] = x_ref[...] * -2.0
        pos_ref[...] = jnp.full((rows, _LANES), -inf, jnp.float32)
        neg_ref[...] = jnp.full((rows, _LANES), inf, jnp.float32)

    y = y_ref[...]
    yy = y * y
    ones8 = jnp.ones((8, edim), jnp.float32)
    y2m = jax.lax.dot_general(
        ones8, yy, (((1,), (1,)), ((), ())), preferred_element_type=jnp.float32
    )
    y2row = y2m[0:1, :]

    m = jax.lax.dot_general(
        xs_ref[...], y, (((1,), (1,)), ((), ())),
        preferred_element_type=jnp.float32,
    )
    t = m + y2row

    n_sub = _COL_BLOCK // _LANES

    @pl.when(j < n_band_blocks)
    def _():
        pos_acc = pos_ref[...]
        neg_acc = neg_ref[...]
        per_band = _LANES // cluster_size
        for k in range(n_sub):
            chunk = t[:, k * _LANES:(k + 1) * _LANES]
            gidx = j * n_sub + k
            rcl = jax.lax.broadcasted_iota(
                jnp.int32, (rows, _LANES), 0) // cluster_size
            gcl = gidx * per_band + jax.lax.broadcasted_iota(
                jnp.int32, (rows, _LANES), 1) // cluster_size
            bm = rcl == gcl
            neg_acc = jnp.minimum(neg_acc, jnp.where(bm, inf, chunk))
            pos_acc = jnp.maximum(pos_acc, jnp.where(bm, chunk, -inf))
        pos_ref[...] = pos_acc
        neg_ref[...] = neg_acc

    @pl.when(j >= n_band_blocks)
    def _():
        neg_acc = neg_ref[...]
        for k in range(n_sub):
            neg_acc = jnp.minimum(neg_acc, t[:, k * _LANES:(k + 1) * _LANES])
        neg_ref[...] = neg_acc

    @pl.when(j == n_col_blocks - 1)
    def _():
        x = x_ref[...]
        xx = x * x
        x2m = jax.lax.dot_general(
            xx, jnp.ones((edim, _LANES), jnp.float32),
            (((1,), (0,)), ((), ())), preferred_element_type=jnp.float32,
        )
        x2 = x2m[:, 0:1]
        pos = jnp.max(pos_ref[...], axis=1, keepdims=True)
        neg = jnp.min(neg_ref[...], axis=1, keepdims=True)
        ap = jnp.sqrt(jnp.maximum(x2 + pos, 0.0))
        an = jnp.sqrt(jnp.maximum(x2 + neg, 0.0))
        out_ref[...] = jnp.sum(jnp.maximum(ap - an + _MARGIN, 0.0)).reshape(1, 1)


def kernel(batch):
    n_clusters, cluster_size, edim = batch.shape
    all_embeds = batch.reshape(-1, edim)
    n_total = all_embeds.shape[0]
    base, rem = divmod(n_clusters, _WORLD_SIZE)
    my_clusters = base + (1 if _RANK < rem else 0)
    my_rows = my_clusters * cluster_size
    n_col_blocks = n_total // _COL_BLOCK
    n_band_blocks = my_rows // _COL_BLOCK

    out = pl.pallas_call(
        functools.partial(_triplet_body, rows=my_rows, edim=edim,
                          cluster_size=cluster_size,
                          n_col_blocks=n_col_blocks,
                          n_band_blocks=n_band_blocks),
        grid=(n_col_blocks,),
        in_specs=[
            pl.BlockSpec((my_rows, edim), lambda j: (0, 0)),
            pl.BlockSpec((_COL_BLOCK, edim), lambda j: (j, 0)),
        ],
        out_specs=pl.BlockSpec((1, 1), lambda j: (0, 0)),
        out_shape=jax.ShapeDtypeStruct((1, 1), jnp.float32),
        scratch_shapes=[
            pltpu.VMEM((my_rows, edim), jnp.float32),
            pltpu.VMEM((my_rows, _LANES), jnp.float32),
            pltpu.VMEM((my_rows, _LANES), jnp.float32),
        ],
        compiler_params=pltpu.CompilerParams(
            dimension_semantics=("arbitrary",),
        ),
    )(all_embeds, all_embeds)
    return out[0, 0] / my_rows


# speedup vs baseline: 136.0037x; 1.0060x over previous
import functools

import jax
import jax.numpy as jnp
from jax.experimental import pallas as pl
from jax.experimental.pallas import tpu as pltpu

_WORLD_SIZE = 8
_RANK = 0
_MARGIN = 1.0
_COL_BLOCK = 512
_LANES = 128


def _triplet_body(x_ref, y_ref, out_ref, xs_ref, y2_ref, pos_ref, neg_ref, *,
                  rows, edim, cluster_size, n_col_blocks, n_band_blocks):
    j = pl.program_id(0)
    inf = jnp.float32(jnp.inf)

    @pl.when(j == 0)
    def _():
        xs_ref[...] = x_ref[...] * -2.0
        pos_ref[...] = jnp.full((rows, _LANES), -inf, jnp.float32)
        neg_ref[...] = jnp.full((rows, _LANES), inf, jnp.float32)
        yy = y_ref[...] * y_ref[...]
        y2_ref[...] = jax.lax.dot_general(
            jnp.ones((8, edim), jnp.float32), yy,
            (((1,), (1,)), ((), ())), preferred_element_type=jnp.float32,
        )

    ycb = y_ref[pl.ds(j * _COL_BLOCK, _COL_BLOCK), :]
    m = jax.lax.dot_general(
        xs_ref[...], ycb, (((1,), (1,)), ((), ())),
        preferred_element_type=jnp.float32,
    )

    n_sub = _COL_BLOCK // _LANES

    @pl.when(j < n_band_blocks)
    def _():
        pos_acc = pos_ref[...]
        neg_acc = neg_ref[...]
        per_band = _LANES // cluster_size
        for k in range(n_sub):
            y2c = y2_ref[0:1, pl.ds(j * _COL_BLOCK + k * _LANES, _LANES)]
            chunk = m[:, k * _LANES:(k + 1) * _LANES] + y2c
            gidx = j * n_sub + k
            rcl = jax.lax.broadcasted_iota(
                jnp.int32, (rows, _LANES), 0) // cluster_size
            gcl = gidx * per_band + jax.lax.broadcasted_iota(
                jnp.int32, (rows, _LANES), 1) // cluster_size
            bm = rcl == gcl
            neg_acc = jnp.minimum(neg_acc, jnp.where(bm, inf, chunk))
            pos_acc = jnp.maximum(pos_acc, jnp.where(bm, chunk, -inf))
        pos_ref[...] = pos_acc
        neg_ref[...] = neg_acc

    @pl.when(j >= n_band_blocks)
    def _():
        neg_acc = neg_ref[...]
        for k in range(n_sub):
            y2c = y2_ref[0:1, pl.ds(j * _COL_BLOCK + k * _LANES, _LANES)]
            neg_acc = jnp.minimum(
                neg_acc, m[:, k * _LANES:(k + 1) * _LANES] + y2c)
        neg_ref[...] = neg_acc

    @pl.when(j == n_col_blocks - 1)
    def _():
        x = x_ref[...]
        xx = x * x
        x2m = jax.lax.dot_general(
            xx, jnp.ones((edim, _LANES), jnp.float32),
            (((1,), (0,)), ((), ())), preferred_element_type=jnp.float32,
        )
        x2 = x2m[:, 0:1]
        pos = jnp.max(pos_ref[...], axis=1, keepdims=True)
        neg = jnp.min(neg_ref[...], axis=1, keepdims=True)
        ap = jnp.sqrt(jnp.maximum(x2 + pos, 0.0))
        an = jnp.sqrt(jnp.maximum(x2 + neg, 0.0))
        out_ref[...] = jnp.sum(jnp.maximum(ap - an + _MARGIN, 0.0)).reshape(1, 1)


def kernel(batch):
    n_clusters, cluster_size, edim = batch.shape
    all_embeds = batch.reshape(-1, edim)
    n_total = all_embeds.shape[0]
    base, rem = divmod(n_clusters, _WORLD_SIZE)
    my_clusters = base + (1 if _RANK < rem else 0)
    my_rows = my_clusters * cluster_size
    n_col_blocks = n_total // _COL_BLOCK
    n_band_blocks = my_rows // _COL_BLOCK

    out = pl.pallas_call(
        functools.partial(_triplet_body, rows=my_rows, edim=edim,
                          cluster_size=cluster_size,
                          n_col_blocks=n_col_blocks,
                          n_band_blocks=n_band_blocks),
        grid=(n_col_blocks,),
        in_specs=[
            pl.BlockSpec((my_rows, edim), lambda j: (0, 0)),
            pl.BlockSpec((n_total, edim), lambda j: (0, 0)),
        ],
        out_specs=pl.BlockSpec((1, 1), lambda j: (0, 0)),
        out_shape=jax.ShapeDtypeStruct((1, 1), jnp.float32),
        scratch_shapes=[
            pltpu.VMEM((my_rows, edim), jnp.float32),
            pltpu.VMEM((8, n_total), jnp.float32),
            pltpu.VMEM((my_rows, _LANES), jnp.float32),
            pltpu.VMEM((my_rows, _LANES), jnp.float32),
        ],
        compiler_params=pltpu.CompilerParams(
            dimension_semantics=("arbitrary",),
        ),
    )(all_embeds, all_embeds)
    return out[0, 0] / my_rows


# speedup vs baseline: 209.2637x; 1.5387x over previous
import functools

import jax
import jax.numpy as jnp
from jax.experimental import pallas as pl
from jax.experimental.pallas import tpu as pltpu

_WORLD_SIZE = 8
_RANK = 0
_MARGIN = 1.0
_COL_BLOCK = 2048
_SUB_BLOCK = 512
_LANES = 128


def _fold_chunk(m, y2blk, base, k, pos_acc, neg_acc, *,
                col0, rows, cluster_size, band):
    y2c = y2blk[0:1, base + k * _LANES:base + (k + 1) * _LANES]
    chunk = m[:, k * _LANES:(k + 1) * _LANES] + y2c
    if band:
        gidx = (col0 + base) // _LANES + k
        per_band = _LANES // cluster_size
        rcl = jax.lax.broadcasted_iota(
            jnp.int32, (rows, _LANES), 0) // cluster_size
        gcl = gidx * per_band + jax.lax.broadcasted_iota(
            jnp.int32, (rows, _LANES), 1) // cluster_size
        bm = rcl == gcl
        inf = jnp.float32(jnp.inf)
        neg_acc = jnp.minimum(neg_acc, jnp.where(bm, inf, chunk))
        pos_acc = jnp.maximum(pos_acc, jnp.where(bm, chunk, -inf))
    else:
        neg_acc = jnp.minimum(neg_acc, chunk)
    return pos_acc, neg_acc


def _triplet_body(x_ref, y_ref, out_ref, xs_ref, pos_ref, neg_ref, *,
                  rows, edim, cluster_size, n_col_blocks):
    j = pl.program_id(0)
    inf = jnp.float32(jnp.inf)
    n_sub = _COL_BLOCK // _SUB_BLOCK
    n_chunk = _SUB_BLOCK // _LANES

    @pl.when(j == 0)
    def _():
        xs_ref[...] = x_ref[...] * -2.0
        pos_ref[...] = jnp.full((rows, _LANES), -inf, jnp.float32)
        neg_ref[...] = jnp.full((rows, _LANES), inf, jnp.float32)

    y = y_ref[...]
    y2blk = jax.lax.dot_general(
        jnp.ones((8, edim), jnp.float32), y * y,
        (((1,), (1,)), ((), ())), preferred_element_type=jnp.float32,
    )

    def sweep(band_cols):
        pos_acc = pos_ref[...]
        neg_acc = neg_ref[...]
        for s in range(n_sub):
            base = s * _SUB_BLOCK
            ycb = y_ref[base:base + _SUB_BLOCK, :]
            m = jax.lax.dot_general(
                xs_ref[...], ycb, (((1,), (1,)), ((), ())),
                preferred_element_type=jnp.float32,
            )
            for k in range(n_chunk):
                band = base + k * _LANES < band_cols
                pos_acc, neg_acc = _fold_chunk(
                    m, y2blk, base, k, pos_acc, neg_acc,
                    col0=0, rows=rows, cluster_size=cluster_size, band=band)
        if band_cols:
            pos_ref[...] = pos_acc
        neg_ref[...] = neg_acc

    @pl.when(j == 0)
    def _():
        sweep(min(rows, _COL_BLOCK))

    @pl.when(j > 0)
    def _():
        sweep(0)

    @pl.when(j == n_col_blocks - 1)
    def _():
        x = x_ref[...]
        xx = x * x
        x2m = jax.lax.dot_general(
            xx, jnp.ones((edim, _LANES), jnp.float32),
            (((1,), (0,)), ((), ())), preferred_element_type=jnp.float32,
        )
        x2 = x2m[:, 0:1]
        pos = jnp.max(pos_ref[...], axis=1, keepdims=True)
        neg = jnp.min(neg_ref[...], axis=1, keepdims=True)
        ap = jnp.sqrt(jnp.maximum(x2 + pos, 0.0))
        an = jnp.sqrt(jnp.maximum(x2 + neg, 0.0))
        out_ref[...] = jnp.sum(jnp.maximum(ap - an + _MARGIN, 0.0)).reshape(1, 1)


def kernel(batch):
    n_clusters, cluster_size, edim = batch.shape
    all_embeds = batch.reshape(-1, edim)
    n_total = all_embeds.shape[0]
    base, rem = divmod(n_clusters, _WORLD_SIZE)
    my_clusters = base + (1 if _RANK < rem else 0)
    my_rows = my_clusters * cluster_size
    n_col_blocks = n_total // _COL_BLOCK

    out = pl.pallas_call(
        functools.partial(_triplet_body, rows=my_rows, edim=edim,
                          cluster_size=cluster_size,
                          n_col_blocks=n_col_blocks),
        grid=(n_col_blocks,),
        in_specs=[
            pl.BlockSpec((my_rows, edim), lambda j: (0, 0)),
            pl.BlockSpec((_COL_BLOCK, edim), lambda j: (j, 0)),
        ],
        out_specs=pl.BlockSpec((1, 1), lambda j: (0, 0)),
        out_shape=jax.ShapeDtypeStruct((1, 1), jnp.float32),
        scratch_shapes=[
            pltpu.VMEM((my_rows, edim), jnp.float32),
            pltpu.VMEM((my_rows, _LANES), jnp.float32),
            pltpu.VMEM((my_rows, _LANES), jnp.float32),
        ],
        compiler_params=pltpu.CompilerParams(
            dimension_semantics=("arbitrary",),
        ),
    )(all_embeds, all_embeds)
    return out[0, 0] / my_rows


# speedup vs baseline: 218.6870x; 1.0450x over previous
import functools

import jax
import jax.numpy as jnp
from jax.experimental import pallas as pl
from jax.experimental.pallas import tpu as pltpu

_WORLD_SIZE = 8
_RANK = 0
_MARGIN = 1.0
_COL_BLOCK = 2048
_SUB_BLOCK = 512
_LANES = 128


def _triplet_body(x_ref, y_ref, out_ref, xs_ref, pos_ref, neg_ref, *,
                  rows, edim, cluster_size, n_col_blocks):
    j = pl.program_id(0)
    inf = jnp.float32(jnp.inf)
    n_sub = _COL_BLOCK // _SUB_BLOCK
    n_chunk = _SUB_BLOCK // _LANES

    @pl.when(j == 0)
    def _():
        xs_ref[...] = x_ref[...] * -2.0
        neg_ref[...] = jnp.full((rows, _LANES), inf, jnp.float32)

    y = y_ref[...]
    y2blk = jax.lax.dot_general(
        jnp.ones((8, edim), jnp.float32), y * y,
        (((1,), (1,)), ((), ())), preferred_element_type=jnp.float32,
    )

    def sweep(band_cols):
        if band_cols:
            li = jax.lax.broadcasted_iota(
                jnp.int32, (_LANES, _LANES), 0) // cluster_size
            ci = jax.lax.broadcasted_iota(
                jnp.int32, (_LANES, _LANES), 1) // cluster_size
            bm = li == ci
            inf_mask = jnp.where(bm, inf, 0.0)
            ninf_mask = jnp.where(bm, 0.0, -inf)
        for s in range(n_sub):
            base = s * _SUB_BLOCK
            ycb = y_ref[base:base + _SUB_BLOCK, :]
            m = jax.lax.dot_general(
                xs_ref[...], ycb, (((1,), (1,)), ((), ())),
                preferred_element_type=jnp.float32,
            )
            for k in range(n_chunk):
                col = base + k * _LANES
                y2c = y2blk[0:1, col:col + _LANES]
                chunk = m[:, k * _LANES:(k + 1) * _LANES] + y2c
                if col < band_cols:
                    r0 = col
                    if r0 > 0:
                        neg_ref[0:r0, :] = jnp.minimum(
                            neg_ref[0:r0, :], chunk[0:r0, :])
                    if r0 + _LANES < rows:
                        neg_ref[r0 + _LANES:rows, :] = jnp.minimum(
                            neg_ref[r0 + _LANES:rows, :],
                            chunk[r0 + _LANES:rows, :])
                    sub = chunk[r0:r0 + _LANES, :]
                    neg_ref[r0:r0 + _LANES, :] = jnp.minimum(
                        neg_ref[r0:r0 + _LANES, :], sub + inf_mask)
                    pos_ref[r0:r0 + _LANES, :] = sub + ninf_mask
                else:
                    neg_ref[...] = jnp.minimum(neg_ref[...], chunk)

    @pl.when(j == 0)
    def _():
        sweep(min(rows, _COL_BLOCK))

    @pl.when(j > 0)
    def _():
        sweep(0)

    @pl.when(j == n_col_blocks - 1)
    def _():
        x = x_ref[...]
        xx = x * x
        x2m = jax.lax.dot_general(
            xx, jnp.ones((edim, _LANES), jnp.float32),
            (((1,), (0,)), ((), ())), preferred_element_type=jnp.float32,
        )
        x2 = x2m[:, 0:1]
        pos = jnp.max(pos_ref[...], axis=1, keepdims=True)
        neg = jnp.min(neg_ref[...], axis=1, keepdims=True)
        ap = jnp.sqrt(jnp.maximum(x2 + pos, 0.0))
        an = jnp.sqrt(jnp.maximum(x2 + neg, 0.0))
        out_ref[...] = jnp.sum(jnp.maximum(ap - an + _MARGIN, 0.0)).reshape(1, 1)


def kernel(batch):
    n_clusters, cluster_size, edim = batch.shape
    all_embeds = batch.reshape(-1, edim)
    n_total = all_embeds.shape[0]
    base, rem = divmod(n_clusters, _WORLD_SIZE)
    my_clusters = base + (1 if _RANK < rem else 0)
    my_rows = my_clusters * cluster_size
    n_col_blocks = n_total // _COL_BLOCK

    out = pl.pallas_call(
        functools.partial(_triplet_body, rows=my_rows, edim=edim,
                          cluster_size=cluster_size,
                          n_col_blocks=n_col_blocks),
        grid=(n_col_blocks,),
        in_specs=[
            pl.BlockSpec((my_rows, edim), lambda j: (0, 0)),
            pl.BlockSpec((_COL_BLOCK, edim), lambda j: (j, 0)),
        ],
        out_specs=pl.BlockSpec((1, 1), lambda j: (0, 0)),
        out_shape=jax.ShapeDtypeStruct((1, 1), jnp.float32),
        scratch_shapes=[
            pltpu.VMEM((my_rows, edim), jnp.float32),
            pltpu.VMEM((my_rows, _LANES), jnp.float32),
            pltpu.VMEM((my_rows, _LANES), jnp.float32),
        ],
        compiler_params=pltpu.CompilerParams(
            dimension_semantics=("arbitrary",),
        ),
    )(all_embeds, all_embeds)
    return out[0, 0] / my_rows
